# baseline (speedup 1.0000x reference)
import functools
import math

import jax
import jax.numpy as jnp
from jax.experimental import pallas as pl
from jax.experimental.pallas import tpu as pltpu

T = 2048
D = 1024
F = 4096
E = 8
K = 2
CAP = int(math.ceil(T * K / E * 1.2))
CP = 640

_NEG = -3.0e38


def _router_body(x_ref, wg_ref, out_ref, oh_ref, excl_ref):
    x = x_ref[...]
    logits = jax.lax.dot_general(x, wg_ref[...], (((1,), (0,)), ((), ())))
    iota = jax.lax.broadcasted_iota(jnp.int32, (T, E), 1)
    m0 = jnp.max(logits, axis=1, keepdims=True)
    a0 = jnp.min(jnp.where(logits == m0, iota, E), axis=1, keepdims=True)
    l1 = jnp.where(iota == a0, _NEG, logits)
    m1 = jnp.max(l1, axis=1, keepdims=True)
    a1 = jnp.min(jnp.where(l1 == m1, iota, E), axis=1, keepdims=True)
    ed = jnp.exp(m1 - m0)
    g0 = 1.0 / (1.0 + ed)
    g1 = 1.0 - g0
    oh_ref[...] = ((iota == a0) | (iota == a1)).astype(jnp.float32)

    chunk = 128
    r = jax.lax.broadcasted_iota(jnp.int32, (chunk, chunk), 0)
    c = jax.lax.broadcasted_iota(jnp.int32, (chunk, chunk), 1)
    ltri = (r > c).astype(jnp.float32)

    def body(i, carry):
        blk = oh_ref[pl.ds(i * chunk, chunk), :]
        excl_ref[pl.ds(i * chunk, chunk), :] = (
            jax.lax.dot_general(ltri, blk, (((1,), (0,)), ((), ()))) + carry
        )
        return carry + jnp.sum(blk, axis=0, keepdims=True)

    jax.lax.fori_loop(0, T // chunk, body, jnp.zeros((1, E), jnp.float32))

    excl = excl_ref[...]
    p0 = jnp.sum(jnp.where(iota == a0, excl, 0.0), axis=1, keepdims=True)
    p1 = jnp.sum(jnp.where(iota == a1, excl, 0.0), axis=1, keepdims=True)
    k0 = (p0 < CAP).astype(jnp.float32)
    k1 = (p1 < CAP).astype(jnp.float32)
    out = jnp.concatenate(
        [
            a0.astype(jnp.float32),
            a1.astype(jnp.float32),
            p0,
            p1,
            g0 * k0,
            g1 * k1,
            k0,
            k1,
        ],
        axis=1,
    )
    out_ref[...] = out


def _dispatch_body(ft_ref, x_ref, buf_ref):
    e = pl.program_id(0).astype(jnp.float32)
    e0 = ft_ref[0:1, :]
    e1 = ft_ref[1:2, :]
    p0 = ft_ref[2:3, :]
    p1 = ft_ref[3:4, :]
    k0 = ft_ref[6:7, :]
    k1 = ft_ref[7:8, :]
    ci = jax.lax.broadcasted_iota(jnp.int32, (CP, T), 0).astype(jnp.float32)
    m0 = ((e0 == e) & (p0 == ci) & (k0 > 0.0)).astype(jnp.bfloat16)
    m1 = ((e1 == e) & (p1 == ci) & (k1 > 0.0)).astype(jnp.bfloat16)
    buf_ref[0] = jax.lax.dot_general(
        m0 + m1, x_ref[...], (((1,), (0,)), ((), ())),
        preferred_element_type=jnp.float32,
    )


def _ffn_body(buf_ref, w1_ref, b1_ref, w2_ref, b2_ref, out_ref):
    f = pl.program_id(1)

    @pl.when(f == 0)
    def _init():
        out_ref[0] = jnp.broadcast_to(b2_ref[0], (CP, D))

    h = jax.lax.dot_general(
        buf_ref[0].astype(jnp.bfloat16), w1_ref[0], (((1,), (0,)), ((), ())),
        preferred_element_type=jnp.float32,
    )
    h = h + b1_ref[0]
    h3 = h * h * h
    g = 0.5 * h * (1.0 + jnp.tanh(0.7978845608028654 * (h + 0.044715 * h3)))
    out_ref[0] += jax.lax.dot_general(
        g.astype(jnp.bfloat16), w2_ref[0], (((1,), (0,)), ((), ())),
        preferred_element_type=jnp.float32,
    )


def _combine_body(f_ref, y_ref, out_ref):
    e = pl.program_id(0).astype(jnp.float32)

    @pl.when(pl.program_id(0) == 0)
    def _init():
        out_ref[...] = jnp.zeros((T, D), jnp.float32)

    e0 = f_ref[:, 0:1]
    e1 = f_ref[:, 1:2]
    p0 = f_ref[:, 2:3]
    p1 = f_ref[:, 3:4]
    w0 = f_ref[:, 4:5]
    w1 = f_ref[:, 5:6]
    ci = jax.lax.broadcasted_iota(jnp.int32, (T, CP), 1).astype(jnp.float32)
    g = w0 * ((e0 == e) & (p0 == ci)).astype(jnp.float32)
    g = g + w1 * ((e1 == e) & (p1 == ci)).astype(jnp.float32)
    out_ref[...] += jax.lax.dot_general(
        g.astype(jnp.bfloat16), y_ref[0].astype(jnp.bfloat16),
        (((1,), (0,)), ((), ())),
        preferred_element_type=jnp.float32,
    )


@jax.jit
def kernel(x, Wg, W1, b1, W2, b2):
    fields = pl.pallas_call(
        _router_body,
        out_shape=jax.ShapeDtypeStruct((T, E), jnp.float32),
        scratch_shapes=[
            pltpu.VMEM((T, E), jnp.float32),
            pltpu.VMEM((T, E), jnp.float32),
        ],
    )(x, Wg)
    ft = fields.T

    buf = pl.pallas_call(
        _dispatch_body,
        grid=(E,),
        in_specs=[
            pl.BlockSpec((E, T), lambda e: (0, 0)),
            pl.BlockSpec((T, D), lambda e: (0, 0)),
        ],
        out_specs=pl.BlockSpec((1, CP, D), lambda e: (e, 0, 0)),
        out_shape=jax.ShapeDtypeStruct((E, CP, D), jnp.float32),
    )(ft, x.astype(jnp.bfloat16))

    fb = 512
    yexp = pl.pallas_call(
        _ffn_body,
        grid=(E, F // fb),
        in_specs=[
            pl.BlockSpec((1, CP, D), lambda e, f: (e, 0, 0)),
            pl.BlockSpec((1, D, fb), lambda e, f: (e, 0, f)),
            pl.BlockSpec((1, 1, fb), lambda e, f: (e, 0, f)),
            pl.BlockSpec((1, fb, D), lambda e, f: (e, f, 0)),
            pl.BlockSpec((1, 1, D), lambda e, f: (e, 0, 0)),
        ],
        out_specs=pl.BlockSpec((1, CP, D), lambda e, f: (e, 0, 0)),
        out_shape=jax.ShapeDtypeStruct((E, CP, D), jnp.float32),
    )(
        buf,
        W1.astype(jnp.bfloat16),
        b1.reshape(E, 1, F),
        W2.astype(jnp.bfloat16),
        b2.reshape(E, 1, D),
    )

    y = pl.pallas_call(
        _combine_body,
        grid=(E,),
        in_specs=[
            pl.BlockSpec((T, E), lambda e: (0, 0)),
            pl.BlockSpec((1, CP, D), lambda e: (e, 0, 0)),
        ],
        out_specs=pl.BlockSpec((T, D), lambda e: (0, 0)),
        out_shape=jax.ShapeDtypeStruct((T, D), jnp.float32),
    )(fields, yexp)
    return y


# speedup vs baseline: 1.3035x; 1.3035x over previous
import functools
import math

import jax
import jax.numpy as jnp
from jax import lax
from jax.experimental import pallas as pl
from jax.experimental.pallas import tpu as pltpu
from jax.experimental.pallas import tpu_sc as plsc

T = 2048
D = 1024
F = 4096
E = 8
K = 2
CAP = int(math.ceil(T * K / E * 1.2))
CP = 640
S = E * CP
NW = 32
SPW = S // NW
TPW = T // NW

_NEG = -3.0e38


def _router_body(x_ref, wg_ref, out_ref, sidx_ref, tokv_ref, oh_ref, excl_ref):
    x = x_ref[...]
    logits = jax.lax.dot_general(x, wg_ref[...], (((1,), (0,)), ((), ())))
    iota = jax.lax.broadcasted_iota(jnp.int32, (T, E), 1)
    m0 = jnp.max(logits, axis=1, keepdims=True)
    a0 = jnp.min(jnp.where(logits == m0, iota, E), axis=1, keepdims=True)
    l1 = jnp.where(iota == a0, _NEG, logits)
    m1 = jnp.max(l1, axis=1, keepdims=True)
    a1 = jnp.min(jnp.where(l1 == m1, iota, E), axis=1, keepdims=True)
    ed = jnp.exp(m1 - m0)
    g0 = 1.0 / (1.0 + ed)
    g1 = 1.0 - g0
    oh_ref[...] = ((iota == a0) | (iota == a1)).astype(jnp.float32)

    chunk = 128
    r = jax.lax.broadcasted_iota(jnp.int32, (chunk, chunk), 0)
    c = jax.lax.broadcasted_iota(jnp.int32, (chunk, chunk), 1)
    ltri = (r > c).astype(jnp.float32)

    def body(i, carry):
        blk = oh_ref[pl.ds(i * chunk, chunk), :]
        excl_ref[pl.ds(i * chunk, chunk), :] = (
            jax.lax.dot_general(ltri, blk, (((1,), (0,)), ((), ()))) + carry
        )
        return carry + jnp.sum(blk, axis=0, keepdims=True)

    jax.lax.fori_loop(0, T // chunk, body, jnp.zeros((1, E), jnp.float32))

    excl = excl_ref[...]
    p0 = jnp.sum(jnp.where(iota == a0, excl, 0.0), axis=1, keepdims=True)
    p1 = jnp.sum(jnp.where(iota == a1, excl, 0.0), axis=1, keepdims=True)
    k0 = (p0 < CAP).astype(jnp.float32)
    k1 = (p1 < CAP).astype(jnp.float32)
    out_ref[...] = jnp.concatenate(
        [
            a0.astype(jnp.float32),
            a1.astype(jnp.float32),
            p0,
            p1,
            g0 * k0,
            g1 * k1,
            k0,
            k1,
        ],
        axis=1,
    )
    dead = jnp.float32(CP - 1)
    s0 = a0.astype(jnp.float32) * CP + jnp.where(k0 > 0.0, p0, dead)
    s1 = a1.astype(jnp.float32) * CP + jnp.where(k1 > 0.0, p1, dead)
    sidx_ref[...] = jnp.concatenate([s0, s1], axis=1).astype(jnp.int32)
    ti = jax.lax.broadcasted_iota(jnp.int32, (T, 1), 0).astype(jnp.float32) + 1.0
    tokv_ref[...] = jnp.concatenate([ti * k0, ti * k1], axis=1).astype(jnp.int32)


def _ffn_body(buf_ref, w1_ref, b1_ref, w2_ref, b2_ref, ws_ref, out_ref):
    f = pl.program_id(1)

    @pl.when(f == 0)
    def _init():
        out_ref[0] = jnp.broadcast_to(b2_ref[0], (CP, D))

    h = jax.lax.dot_general(buf_ref[0], w1_ref[0], (((1,), (0,)), ((), ())))
    h = h + b1_ref[0]
    h3 = h * h * h
    g = 0.5 * h * (1.0 + jnp.tanh(0.7978845608028654 * (h + 0.044715 * h3)))
    out_ref[0] += jax.lax.dot_general(g, w2_ref[0], (((1,), (0,)), ((), ())))

    @pl.when(f == F // 512 - 1)
    def _scale():
        out_ref[0] = out_ref[0] * ws_ref[0]


def _sc_dispatch_body(
    sidx_h, tokv_h, wv_h, xz_h, buf_h, wslot_h,
    asg_i, asg_v, asg_w, stok, sw, idx_v, rows, sh_tok, sh_w,
):
    cid = lax.axis_index("c")
    sid = lax.axis_index("s")

    @pl.when(sid == 0)
    def _build():
        @pl.loop(0, S // 16, unroll=8)
        def _zero(i):
            stok[pl.ds(i * 16, 16)] = jnp.zeros((16,), jnp.int32)
            sw[pl.ds(i * 16, 16)] = jnp.zeros((16,), jnp.float32)

        pltpu.sync_copy(sidx_h, asg_i)
        pltpu.sync_copy(tokv_h, asg_v)
        pltpu.sync_copy(wv_h, asg_w)

        @pl.loop(0, (T * K) // 16, unroll=4)
        def _scatter(i):
            sl = pl.ds(i * 16, 16)
            idx = asg_i[sl]
            plsc.store_scatter(stok, [idx], asg_v[sl])
            plsc.store_scatter(sw, [idx], asg_w[sl])

        pltpu.sync_copy(stok, sh_tok)
        pltpu.sync_copy(sw, sh_w)

    @pl.when((sid == 0) & (cid == 0))
    def _emit_w():
        pltpu.sync_copy(sw, wslot_h)

    plsc.subcore_barrier()

    wid = sid * 2 + cid
    base = wid * SPW
    pltpu.sync_copy(sh_tok.at[pl.ds(base, SPW)], idx_v)
    for k in range(2):
        half = SPW // 2
        pltpu.sync_copy(
            xz_h.at[idx_v.at[pl.ds(k * half, half)]],
            rows,
        )
        pltpu.sync_copy(rows, buf_h.at[pl.ds(base + k * half, half)])


def _sc_combine_body(s0_h, s1_h, ys_h, y_h, i0, i1, r0, r1):
    cid = lax.axis_index("c")
    sid = lax.axis_index("s")
    wid = sid * 2 + cid
    for chunk in range(2):
        base = wid * TPW + chunk * (TPW // 2)
        n = TPW // 2
        pltpu.sync_copy(s0_h.at[pl.ds(base, n)], i0)
        pltpu.sync_copy(s1_h.at[pl.ds(base, n)], i1)
        pltpu.sync_copy(ys_h.at[i0], r0)
        pltpu.sync_copy(ys_h.at[i1], r1)

        @pl.loop(0, n)
        def _add(i):
            for j in range(D // 16):
                sl = pl.ds(j * 16, 16)
                r0[i, sl] = r0[i, sl] + r1[i, sl]

        pltpu.sync_copy(r0, y_h.at[pl.ds(base, n)])


_SC_MESH = plsc.VectorSubcoreMesh(core_axis_name="c", subcore_axis_name="s")

_sc_dispatch = functools.partial(
    pl.kernel,
    mesh=_SC_MESH,
    compiler_params=pltpu.CompilerParams(needs_layout_passes=False),
    out_type=(
        jax.ShapeDtypeStruct((S, D), jnp.float32),
        jax.ShapeDtypeStruct((S,), jnp.float32),
    ),
    scratch_types=[
        pltpu.VMEM((T * K,), jnp.int32),
        pltpu.VMEM((T * K,), jnp.int32),
        pltpu.VMEM((T * K,), jnp.float32),
        pltpu.VMEM((S,), jnp.int32),
        pltpu.VMEM((S,), jnp.float32),
        pltpu.VMEM((SPW,), jnp.int32),
        pltpu.VMEM((SPW // 2, D), jnp.float32),
        pltpu.VMEM_SHARED((S,), jnp.int32),
        pltpu.VMEM_SHARED((S,), jnp.float32),
    ],
)(_sc_dispatch_body)

_sc_combine = functools.partial(
    pl.kernel,
    mesh=_SC_MESH,
    out_type=jax.ShapeDtypeStruct((T, D), jnp.float32),
    scratch_types=[
        pltpu.VMEM((TPW // 2,), jnp.int32),
        pltpu.VMEM((TPW // 2,), jnp.int32),
        pltpu.VMEM((TPW // 2, D), jnp.float32),
        pltpu.VMEM((TPW // 2, D), jnp.float32),
    ],
)(_sc_combine_body)


@jax.jit
def kernel(x, Wg, W1, b1, W2, b2):
    fields, sidx, tokv = pl.pallas_call(
        _router_body,
        out_shape=(
            jax.ShapeDtypeStruct((T, E), jnp.float32),
            jax.ShapeDtypeStruct((T, K), jnp.int32),
            jax.ShapeDtypeStruct((T, K), jnp.int32),
        ),
        scratch_shapes=[
            pltpu.VMEM((T, E), jnp.float32),
            pltpu.VMEM((T, E), jnp.float32),
        ],
    )(x, Wg)

    xz = jnp.concatenate([jnp.zeros((1, D), x.dtype), x], axis=0)
    buf_flat, wslot = _sc_dispatch(
        sidx.reshape(T * K),
        tokv.reshape(T * K),
        fields[:, 4:6].reshape(T * K),
        xz,
    )

    fb = 512
    yexp = pl.pallas_call(
        _ffn_body,
        grid=(E, F // fb),
        in_specs=[
            pl.BlockSpec((1, CP, D), lambda e, f: (e, 0, 0)),
            pl.BlockSpec((1, D, fb), lambda e, f: (e, 0, f)),
            pl.BlockSpec((1, 1, fb), lambda e, f: (e, 0, f)),
            pl.BlockSpec((1, fb, D), lambda e, f: (e, f, 0)),
            pl.BlockSpec((1, 1, D), lambda e, f: (e, 0, 0)),
            pl.BlockSpec((1, CP, 1), lambda e, f: (e, 0, 0)),
        ],
        out_specs=pl.BlockSpec((1, CP, D), lambda e, f: (e, 0, 0)),
        out_shape=jax.ShapeDtypeStruct((E, CP, D), jnp.float32),
    )(
        buf_flat.reshape(E, CP, D),
        W1,
        b1.reshape(E, 1, F),
        W2,
        b2.reshape(E, 1, D),
        wslot.reshape(E, CP, 1),
    )

    y = _sc_combine(
        sidx[:, 0].reshape(T),
        sidx[:, 1].reshape(T),
        yexp.reshape(S, D),
    )
    return y


# speedup vs baseline: 1.4414x; 1.1058x over previous
import functools
import math

import jax
import jax.numpy as jnp
from jax import lax
from jax.experimental import pallas as pl
from jax.experimental.pallas import tpu as pltpu
from jax.experimental.pallas import tpu_sc as plsc

T = 2048
D = 1024
F = 4096
E = 8
K = 2
CAP = int(math.ceil(T * K / E * 1.2))
CP = 640
S = E * CP
NW = 32
SPW = S // NW
TPW = T // NW

_NEG = -3.0e38


def _router_body(x_ref, wg_ref, out_ref, sidx_ref, oh_ref, excl_ref):
    x = x_ref[...]
    logits = jax.lax.dot_general(x, wg_ref[...], (((1,), (0,)), ((), ())))
    iota = jax.lax.broadcasted_iota(jnp.int32, (T, E), 1)
    m0 = jnp.max(logits, axis=1, keepdims=True)
    a0 = jnp.min(jnp.where(logits == m0, iota, E), axis=1, keepdims=True)
    l1 = jnp.where(iota == a0, _NEG, logits)
    m1 = jnp.max(l1, axis=1, keepdims=True)
    a1 = jnp.min(jnp.where(l1 == m1, iota, E), axis=1, keepdims=True)
    ed = jnp.exp(m1 - m0)
    g0 = 1.0 / (1.0 + ed)
    g1 = 1.0 - g0
    oh_ref[...] = ((iota == a0) | (iota == a1)).astype(jnp.float32)

    chunk = 128
    r = jax.lax.broadcasted_iota(jnp.int32, (chunk, chunk), 0)
    c = jax.lax.broadcasted_iota(jnp.int32, (chunk, chunk), 1)
    ltri = (r > c).astype(jnp.float32)

    def body(i, carry):
        blk = oh_ref[pl.ds(i * chunk, chunk), :]
        excl_ref[pl.ds(i * chunk, chunk), :] = (
            jax.lax.dot_general(ltri, blk, (((1,), (0,)), ((), ()))) + carry
        )
        return carry + jnp.sum(blk, axis=0, keepdims=True)

    jax.lax.fori_loop(0, T // chunk, body, jnp.zeros((1, E), jnp.float32))

    excl = excl_ref[...]
    p0 = jnp.sum(jnp.where(iota == a0, excl, 0.0), axis=1, keepdims=True)
    p1 = jnp.sum(jnp.where(iota == a1, excl, 0.0), axis=1, keepdims=True)
    k0 = (p0 < CAP).astype(jnp.float32)
    k1 = (p1 < CAP).astype(jnp.float32)
    out_ref[...] = jnp.concatenate(
        [
            a0.astype(jnp.float32),
            a1.astype(jnp.float32),
            p0,
            p1,
            g0 * k0,
            g1 * k1,
            k0,
            k1,
        ],
        axis=1,
    )
    dead = jnp.float32(CP - 1)
    s0 = a0.astype(jnp.float32) * CP + jnp.where(k0 > 0.0, p0, dead)
    s1 = a1.astype(jnp.float32) * CP + jnp.where(k1 > 0.0, p1, dead)
    sidx_ref[...] = jnp.concatenate([s0, s1], axis=1).astype(jnp.int32)


def _ffn_body(buf_ref, w1_ref, b1_ref, w2_ref, b2_ref, ws_ref, out_ref):
    f = pl.program_id(1)

    @pl.when(f == 0)
    def _init():
        out_ref[0] = jnp.broadcast_to(b2_ref[0], (CP, D))

    h = jax.lax.dot_general(buf_ref[0], w1_ref[0], (((1,), (0,)), ((), ())))
    h = h + b1_ref[0]
    h3 = h * h * h
    g = 0.5 * h * (1.0 + jnp.tanh(0.7978845608028654 * (h + 0.044715 * h3)))
    out_ref[0] += jax.lax.dot_general(g, w2_ref[0], (((1,), (0,)), ((), ())))

    @pl.when(f == F // 512 - 1)
    def _scale():
        out_ref[0] = out_ref[0] * ws_ref[0]


def _dispatch_body(ft_ref, f_ref, x_ref, buf_ref, ws_ref):
    e = pl.program_id(0).astype(jnp.float32)
    e0 = ft_ref[0:1, :]
    e1 = ft_ref[1:2, :]
    p0 = ft_ref[2:3, :]
    p1 = ft_ref[3:4, :]
    k0 = ft_ref[6:7, :]
    k1 = ft_ref[7:8, :]
    ci = jax.lax.broadcasted_iota(jnp.int32, (CP, T), 0).astype(jnp.float32)
    m0 = ((e0 == e) & (p0 == ci) & (k0 > 0.0)).astype(jnp.float32)
    m1 = ((e1 == e) & (p1 == ci) & (k1 > 0.0)).astype(jnp.float32)
    buf_ref[0] = jax.lax.dot_general(
        m0 + m1, x_ref[...], (((1,), (0,)), ((), ()))
    )
    w01 = f_ref[:, 4:6]
    ws = jax.lax.dot_general(m0, w01, (((1,), (0,)), ((), ())))[:, 0:1]
    ws = ws + jax.lax.dot_general(m1, w01, (((1,), (0,)), ((), ())))[:, 1:2]
    ws_ref[0] = ws


def _sc_combine_body(s0_h, s1_h, ys_h, y_h, i0, i1, r0, r1):
    cid = lax.axis_index("c")
    sid = lax.axis_index("s")
    wid = sid * 2 + cid
    for chunk in range(2):
        base = wid * TPW + chunk * (TPW // 2)
        n = TPW // 2
        pltpu.sync_copy(s0_h.at[pl.ds(base, n)], i0)
        pltpu.sync_copy(s1_h.at[pl.ds(base, n)], i1)
        pltpu.sync_copy(ys_h.at[i0], r0)
        pltpu.sync_copy(ys_h.at[i1], r1)

        @pl.loop(0, n)
        def _add(i):
            for j in range(D // 16):
                sl = pl.ds(j * 16, 16)
                r0[i, sl] = r0[i, sl] + r1[i, sl]

        pltpu.sync_copy(r0, y_h.at[pl.ds(base, n)])


_SC_MESH = plsc.VectorSubcoreMesh(core_axis_name="c", subcore_axis_name="s")

_sc_combine = functools.partial(
    pl.kernel,
    mesh=_SC_MESH,
    out_type=jax.ShapeDtypeStruct((T, D), jnp.float32),
    scratch_types=[
        pltpu.VMEM((TPW // 2,), jnp.int32),
        pltpu.VMEM((TPW // 2,), jnp.int32),
        pltpu.VMEM((TPW // 2, D), jnp.float32),
        pltpu.VMEM((TPW // 2, D), jnp.float32),
    ],
)(_sc_combine_body)


@jax.jit
def kernel(x, Wg, W1, b1, W2, b2):
    fields, sidx = pl.pallas_call(
        _router_body,
        out_shape=(
            jax.ShapeDtypeStruct((T, E), jnp.float32),
            jax.ShapeDtypeStruct((T, K), jnp.int32),
        ),
        scratch_shapes=[
            pltpu.VMEM((T, E), jnp.float32),
            pltpu.VMEM((T, E), jnp.float32),
        ],
    )(x, Wg)

    ft = fields.T
    buf, ws3 = pl.pallas_call(
        _dispatch_body,
        grid=(E,),
        in_specs=[
            pl.BlockSpec((E, T), lambda e: (0, 0)),
            pl.BlockSpec((T, E), lambda e: (0, 0)),
            pl.BlockSpec((T, D), lambda e: (0, 0)),
        ],
        out_specs=(
            pl.BlockSpec((1, CP, D), lambda e: (e, 0, 0)),
            pl.BlockSpec((1, CP, 1), lambda e: (e, 0, 0)),
        ),
        out_shape=(
            jax.ShapeDtypeStruct((E, CP, D), jnp.float32),
            jax.ShapeDtypeStruct((E, CP, 1), jnp.float32),
        ),
    )(ft, fields, x)

    fb = 512
    yexp = pl.pallas_call(
        _ffn_body,
        grid=(E, F // fb),
        in_specs=[
            pl.BlockSpec((1, CP, D), lambda e, f: (e, 0, 0)),
            pl.BlockSpec((1, D, fb), lambda e, f: (e, 0, f)),
            pl.BlockSpec((1, 1, fb), lambda e, f: (e, 0, f)),
            pl.BlockSpec((1, fb, D), lambda e, f: (e, f, 0)),
            pl.BlockSpec((1, 1, D), lambda e, f: (e, 0, 0)),
            pl.BlockSpec((1, CP, 1), lambda e, f: (e, 0, 0)),
        ],
        out_specs=pl.BlockSpec((1, CP, D), lambda e, f: (e, 0, 0)),
        out_shape=jax.ShapeDtypeStruct((E, CP, D), jnp.float32),
    )(buf, W1, b1.reshape(E, 1, F), W2, b2.reshape(E, 1, D), ws3)

    y = _sc_combine(
        sidx[:, 0].reshape(T),
        sidx[:, 1].reshape(T),
        yexp.reshape(S, D),
    )
    return y


# speedup vs baseline: 1.5973x; 1.1082x over previous
import functools
import math

import jax
import jax.numpy as jnp
from jax.experimental import pallas as pl
from jax.experimental.pallas import tpu as pltpu

T = 2048
D = 1024
F = 4096
E = 8
K = 2
CAP = int(math.ceil(T * K / E * 1.2))
CP = 640
FB = 512
NF = F // FB

_NEG = -3.0e38


def _router_body(x_ref, wg_ref, out_ref, oh_ref, excl_ref):
    x = x_ref[...]
    logits = jax.lax.dot_general(x, wg_ref[...], (((1,), (0,)), ((), ())))
    iota = jax.lax.broadcasted_iota(jnp.int32, (T, E), 1)
    m0 = jnp.max(logits, axis=1, keepdims=True)
    a0 = jnp.min(jnp.where(logits == m0, iota, E), axis=1, keepdims=True)
    l1 = jnp.where(iota == a0, _NEG, logits)
    m1 = jnp.max(l1, axis=1, keepdims=True)
    a1 = jnp.min(jnp.where(l1 == m1, iota, E), axis=1, keepdims=True)
    ed = jnp.exp(m1 - m0)
    g0 = 1.0 / (1.0 + ed)
    g1 = 1.0 - g0
    oh_ref[...] = ((iota == a0) | (iota == a1)).astype(jnp.float32)

    chunk = 128
    r = jax.lax.broadcasted_iota(jnp.int32, (chunk, chunk), 0)
    c = jax.lax.broadcasted_iota(jnp.int32, (chunk, chunk), 1)
    ltri = (r > c).astype(jnp.float32)

    def body(i, carry):
        blk = oh_ref[pl.ds(i * chunk, chunk), :]
        excl_ref[pl.ds(i * chunk, chunk), :] = (
            jax.lax.dot_general(ltri, blk, (((1,), (0,)), ((), ()))) + carry
        )
        return carry + jnp.sum(blk, axis=0, keepdims=True)

    jax.lax.fori_loop(0, T // chunk, body, jnp.zeros((1, E), jnp.float32))

    excl = excl_ref[...]
    p0 = jnp.sum(jnp.where(iota == a0, excl, 0.0), axis=1, keepdims=True)
    p1 = jnp.sum(jnp.where(iota == a1, excl, 0.0), axis=1, keepdims=True)
    k0 = (p0 < CAP).astype(jnp.float32)
    k1 = (p1 < CAP).astype(jnp.float32)
    out_ref[...] = jnp.concatenate(
        [
            a0.astype(jnp.float32),
            a1.astype(jnp.float32),
            p0,
            p1,
            g0 * k0,
            g1 * k1,
            k0,
            k1,
        ],
        axis=1,
    )


def _moe_body(ft_ref, f_ref, x_ref, w1_ref, b1_ref, w2_ref, b2_ref,
              y_ref, buf_s, ye_s):
    e = pl.program_id(0)
    f = pl.program_id(1)
    ef = e.astype(jnp.float32)

    @pl.when((e == 0) & (f == 0))
    def _zero_y():
        y_ref[...] = jnp.zeros((T, D), jnp.float32)

    @pl.when(f == 0)
    def _dispatch():
        e0 = ft_ref[0:1, :]
        e1 = ft_ref[1:2, :]
        p0 = ft_ref[2:3, :]
        p1 = ft_ref[3:4, :]
        k0 = ft_ref[6:7, :]
        k1 = ft_ref[7:8, :]
        ci = jax.lax.broadcasted_iota(jnp.int32, (CP, T), 0).astype(jnp.float32)
        m0 = ((e0 == ef) & (p0 == ci) & (k0 > 0.0)).astype(jnp.float32)
        m1 = ((e1 == ef) & (p1 == ci) & (k1 > 0.0)).astype(jnp.float32)
        buf_s[...] = jax.lax.dot_general(
            m0 + m1, x_ref[...], (((1,), (0,)), ((), ()))
        )
        ye_s[...] = jnp.broadcast_to(b2_ref[0], (CP, D))

    h = jax.lax.dot_general(buf_s[...], w1_ref[0], (((1,), (0,)), ((), ())))
    h = h + b1_ref[0]
    h3 = h * h * h
    g = 0.5 * h * (1.0 + jnp.tanh(0.7978845608028654 * (h + 0.044715 * h3)))
    ye_s[...] += jax.lax.dot_general(g, w2_ref[0], (((1,), (0,)), ((), ())))

    @pl.when(f == NF - 1)
    def _combine():
        e0 = f_ref[:, 0:1]
        e1 = f_ref[:, 1:2]
        p0 = f_ref[:, 2:3]
        p1 = f_ref[:, 3:4]
        w0 = f_ref[:, 4:5]
        w1 = f_ref[:, 5:6]
        ci = jax.lax.broadcasted_iota(jnp.int32, (T, CP), 1).astype(jnp.float32)
        gm = w0 * ((e0 == ef) & (p0 == ci)).astype(jnp.float32)
        gm = gm + w1 * ((e1 == ef) & (p1 == ci)).astype(jnp.float32)
        y_ref[...] += jax.lax.dot_general(
            gm, ye_s[...], (((1,), (0,)), ((), ()))
        )


@jax.jit
def kernel(x, Wg, W1, b1, W2, b2):
    fields = pl.pallas_call(
        _router_body,
        out_shape=jax.ShapeDtypeStruct((T, E), jnp.float32),
        scratch_shapes=[
            pltpu.VMEM((T, E), jnp.float32),
            pltpu.VMEM((T, E), jnp.float32),
        ],
    )(x, Wg)
    ft = fields.T

    y = pl.pallas_call(
        _moe_body,
        grid=(E, NF),
        in_specs=[
            pl.BlockSpec((E, T), lambda e, f: (0, 0)),
            pl.BlockSpec((T, E), lambda e, f: (0, 0)),
            pl.BlockSpec((T, D), lambda e, f: (0, 0)),
            pl.BlockSpec((1, D, FB), lambda e, f: (e, 0, f)),
            pl.BlockSpec((1, 1, FB), lambda e, f: (e, 0, f)),
            pl.BlockSpec((1, FB, D), lambda e, f: (e, f, 0)),
            pl.BlockSpec((1, 1, D), lambda e, f: (e, 0, 0)),
        ],
        out_specs=pl.BlockSpec((T, D), lambda e, f: (0, 0)),
        out_shape=jax.ShapeDtypeStruct((T, D), jnp.float32),
        scratch_shapes=[
            pltpu.VMEM((CP, D), jnp.float32),
            pltpu.VMEM((CP, D), jnp.float32),
        ],
    )(ft, fields, x, W1, b1.reshape(E, 1, F), W2, b2.reshape(E, 1, D))
    return y


# speedup vs baseline: 1.8551x; 1.1614x over previous
import functools
import math

import jax
import jax.numpy as jnp
from jax.experimental import pallas as pl
from jax.experimental.pallas import tpu as pltpu

T = 2048
D = 1024
F = 4096
E = 8
K = 2
CAP = int(math.ceil(T * K / E * 1.2))
CP = 640
FB = 1024
NF = F // FB

_NEG = -3.0e38


def _router_body(x_ref, wg_ref, out_ref, oh_ref, excl_ref):
    x = x_ref[...]
    logits = jax.lax.dot_general(x, wg_ref[...], (((1,), (0,)), ((), ())))
    iota = jax.lax.broadcasted_iota(jnp.int32, (T, E), 1)
    m0 = jnp.max(logits, axis=1, keepdims=True)
    a0 = jnp.min(jnp.where(logits == m0, iota, E), axis=1, keepdims=True)
    l1 = jnp.where(iota == a0, _NEG, logits)
    m1 = jnp.max(l1, axis=1, keepdims=True)
    a1 = jnp.min(jnp.where(l1 == m1, iota, E), axis=1, keepdims=True)
    ed = jnp.exp(m1 - m0)
    g0 = 1.0 / (1.0 + ed)
    g1 = 1.0 - g0
    oh_ref[...] = ((iota == a0) | (iota == a1)).astype(jnp.float32)

    chunk = 128
    r = jax.lax.broadcasted_iota(jnp.int32, (chunk, chunk), 0)
    c = jax.lax.broadcasted_iota(jnp.int32, (chunk, chunk), 1)
    ltri = (r > c).astype(jnp.float32)

    def body(i, carry):
        blk = oh_ref[pl.ds(i * chunk, chunk), :]
        excl_ref[pl.ds(i * chunk, chunk), :] = (
            jax.lax.dot_general(ltri, blk, (((1,), (0,)), ((), ()))) + carry
        )
        return carry + jnp.sum(blk, axis=0, keepdims=True)

    jax.lax.fori_loop(0, T // chunk, body, jnp.zeros((1, E), jnp.float32))

    excl = excl_ref[...]
    p0 = jnp.sum(jnp.where(iota == a0, excl, 0.0), axis=1, keepdims=True)
    p1 = jnp.sum(jnp.where(iota == a1, excl, 0.0), axis=1, keepdims=True)
    k0 = (p0 < CAP).astype(jnp.float32)
    k1 = (p1 < CAP).astype(jnp.float32)
    af0 = a0.astype(jnp.float32)
    af1 = a1.astype(jnp.float32)
    q0 = jnp.where(k0 > 0.0, af0 * CP + p0, -1.0)
    q1 = jnp.where(k1 > 0.0, af1 * CP + p1, -1.0)
    z = jnp.zeros((T, 1), jnp.float32)
    out_ref[...] = jnp.concatenate(
        [q0, q1, g0 * k0, g1 * k1, z, z, z, z], axis=1
    )


def _moe_body(ft_ref, f_ref, x_ref, w1_ref, b1_ref, w2_ref, b2_ref,
              y_ref, buf_s, ye_s):
    e = pl.program_id(0)
    f = pl.program_id(1)
    ef = e.astype(jnp.float32)

    @pl.when(f == 0)
    def _dispatch():
        q0 = ft_ref[0:1, :]
        q1 = ft_ref[1:2, :]
        gci = jax.lax.broadcasted_iota(jnp.int32, (CP, T), 0).astype(jnp.float32)
        gci = gci + ef * CP
        m = ((q0 == gci) | (q1 == gci)).astype(jnp.float32)
        buf_s[...] = jax.lax.dot_general(
            m, x_ref[...], (((1,), (0,)), ((), ()))
        )
        ye_s[...] = jnp.broadcast_to(b2_ref[0], (CP, D))

    h = jax.lax.dot_general(buf_s[...], w1_ref[0], (((1,), (0,)), ((), ())))
    h = h + b1_ref[0]
    h3 = h * h * h
    g = 0.5 * h * (1.0 + jnp.tanh(0.7978845608028654 * (h + 0.044715 * h3)))
    ye_s[...] += jax.lax.dot_general(g, w2_ref[0], (((1,), (0,)), ((), ())))

    @pl.when(f == NF - 1)
    def _combine():
        q0 = f_ref[:, 0:1]
        q1 = f_ref[:, 1:2]
        w0 = f_ref[:, 2:3]
        w1 = f_ref[:, 3:4]
        gci = jax.lax.broadcasted_iota(jnp.int32, (T, CP), 1).astype(jnp.float32)
        gci = gci + ef * CP
        gm = w0 * (q0 == gci).astype(jnp.float32)
        gm = gm + w1 * (q1 == gci).astype(jnp.float32)
        part = jax.lax.dot_general(gm, ye_s[...], (((1,), (0,)), ((), ())))

        @pl.when(e == 0)
        def _set():
            y_ref[...] = part

        @pl.when(e > 0)
        def _acc():
            y_ref[...] += part


@jax.jit
def kernel(x, Wg, W1, b1, W2, b2):
    fields = pl.pallas_call(
        _router_body,
        out_shape=jax.ShapeDtypeStruct((T, E), jnp.float32),
        scratch_shapes=[
            pltpu.VMEM((T, E), jnp.float32),
            pltpu.VMEM((T, E), jnp.float32),
        ],
    )(x, Wg)
    ft = fields.T

    y = pl.pallas_call(
        _moe_body,
        grid=(E, NF),
        in_specs=[
            pl.BlockSpec((E, T), lambda e, f: (0, 0)),
            pl.BlockSpec((T, E), lambda e, f: (0, 0)),
            pl.BlockSpec((T, D), lambda e, f: (0, 0)),
            pl.BlockSpec((1, D, FB), lambda e, f: (e, 0, f)),
            pl.BlockSpec((1, 1, FB), lambda e, f: (e, 0, f)),
            pl.BlockSpec((1, FB, D), lambda e, f: (e, f, 0)),
            pl.BlockSpec((1, 1, D), lambda e, f: (e, 0, 0)),
        ],
        out_specs=pl.BlockSpec((T, D), lambda e, f: (0, 0)),
        out_shape=jax.ShapeDtypeStruct((T, D), jnp.float32),
        scratch_shapes=[
            pltpu.VMEM((CP, D), jnp.float32),
            pltpu.VMEM((CP, D), jnp.float32),
        ],
    )(ft, fields, x, W1, b1.reshape(E, 1, F), W2, b2.reshape(E, 1, D))
    return y


# speedup vs baseline: 1.9151x; 1.0324x over previous
import functools
import math

import jax
import jax.numpy as jnp
from jax.experimental import pallas as pl
from jax.experimental.pallas import tpu as pltpu

T = 2048
D = 1024
F = 4096
E = 8
K = 2
CAP = int(math.ceil(T * K / E * 1.2))
CP = 616
FB = 1024
NF = F // FB

_NEG = -3.0e38


def _router_body(x_ref, wg_ref, out_ref, oh_ref, excl_ref):
    x = x_ref[...]
    logits = jax.lax.dot_general(x, wg_ref[...], (((1,), (0,)), ((), ())))
    iota = jax.lax.broadcasted_iota(jnp.int32, (T, E), 1)
    m0 = jnp.max(logits, axis=1, keepdims=True)
    a0 = jnp.min(jnp.where(logits == m0, iota, E), axis=1, keepdims=True)
    l1 = jnp.where(iota == a0, _NEG, logits)
    m1 = jnp.max(l1, axis=1, keepdims=True)
    a1 = jnp.min(jnp.where(l1 == m1, iota, E), axis=1, keepdims=True)
    ed = jnp.exp(m1 - m0)
    g0 = 1.0 / (1.0 + ed)
    g1 = 1.0 - g0
    oh_ref[...] = ((iota == a0) | (iota == a1)).astype(jnp.float32)

    chunk = 256
    r = jax.lax.broadcasted_iota(jnp.int32, (chunk, chunk), 0)
    c = jax.lax.broadcasted_iota(jnp.int32, (chunk, chunk), 1)
    ltri = (r > c).astype(jnp.float32)

    def body(i, carry):
        blk = oh_ref[pl.ds(i * chunk, chunk), :]
        excl_ref[pl.ds(i * chunk, chunk), :] = (
            jax.lax.dot_general(ltri, blk, (((1,), (0,)), ((), ()))) + carry
        )
        return carry + jnp.sum(blk, axis=0, keepdims=True)

    jax.lax.fori_loop(0, T // chunk, body, jnp.zeros((1, E), jnp.float32))

    excl = excl_ref[...]
    p0 = jnp.sum(jnp.where(iota == a0, excl, 0.0), axis=1, keepdims=True)
    p1 = jnp.sum(jnp.where(iota == a1, excl, 0.0), axis=1, keepdims=True)
    k0 = (p0 < CAP).astype(jnp.float32)
    k1 = (p1 < CAP).astype(jnp.float32)
    af0 = a0.astype(jnp.float32)
    af1 = a1.astype(jnp.float32)
    q0 = jnp.where(k0 > 0.0, af0 * CP + p0, -1.0)
    q1 = jnp.where(k1 > 0.0, af1 * CP + p1, -1.0)
    z = jnp.zeros((T, 1), jnp.float32)
    out_ref[...] = jnp.concatenate(
        [q0, q1, g0 * k0, g1 * k1, z, z, z, z], axis=1
    )


def _moe_body(ft_ref, f_ref, x_ref, w1_ref, b1_ref, w2_ref, b2_ref,
              y_ref, buf_s, ye_s):
    e = pl.program_id(0)
    f = pl.program_id(1)
    ef = e.astype(jnp.float32)

    @pl.when(f == 0)
    def _dispatch():
        q0 = ft_ref[0:1, :]
        q1 = ft_ref[1:2, :]
        gci = jax.lax.broadcasted_iota(jnp.int32, (CP, T), 0).astype(jnp.float32)
        gci = gci + ef * CP
        m = ((q0 == gci) | (q1 == gci)).astype(jnp.float32)
        buf_s[...] = jax.lax.dot_general(
            m, x_ref[...], (((1,), (0,)), ((), ()))
        )
        ye_s[...] = jnp.broadcast_to(b2_ref[0], (CP, D))

    h = jax.lax.dot_general(buf_s[...], w1_ref[0], (((1,), (0,)), ((), ())))
    h = h + b1_ref[0]
    h3 = h * h * h
    g = 0.5 * h * (1.0 + jnp.tanh(0.7978845608028654 * (h + 0.044715 * h3)))
    ye_s[...] += jax.lax.dot_general(g, w2_ref[0], (((1,), (0,)), ((), ())))

    @pl.when(f == NF - 1)
    def _combine():
        q0 = f_ref[:, 0:1]
        q1 = f_ref[:, 1:2]
        w0 = f_ref[:, 2:3]
        w1 = f_ref[:, 3:4]
        gci = jax.lax.broadcasted_iota(jnp.int32, (T, CP), 1).astype(jnp.float32)
        gci = gci + ef * CP
        gm = w0 * (q0 == gci).astype(jnp.float32)
        gm = gm + w1 * (q1 == gci).astype(jnp.float32)
        part = jax.lax.dot_general(gm, ye_s[...], (((1,), (0,)), ((), ())))

        @pl.when(e == 0)
        def _set():
            y_ref[...] = part

        @pl.when(e > 0)
        def _acc():
            y_ref[...] += part


@jax.jit
def kernel(x, Wg, W1, b1, W2, b2):
    fields = pl.pallas_call(
        _router_body,
        out_shape=jax.ShapeDtypeStruct((T, E), jnp.float32),
        scratch_shapes=[
            pltpu.VMEM((T, E), jnp.float32),
            pltpu.VMEM((T, E), jnp.float32),
        ],
    )(x, Wg)
    ft = fields.T

    y = pl.pallas_call(
        _moe_body,
        grid=(E, NF),
        in_specs=[
            pl.BlockSpec((E, T), lambda e, f: (0, 0)),
            pl.BlockSpec((T, E), lambda e, f: (0, 0)),
            pl.BlockSpec((T, D), lambda e, f: (0, 0)),
            pl.BlockSpec((1, D, FB), lambda e, f: (e, 0, f)),
            pl.BlockSpec((1, 1, FB), lambda e, f: (e, 0, f)),
            pl.BlockSpec((1, FB, D), lambda e, f: (e, f, 0)),
            pl.BlockSpec((1, 1, D), lambda e, f: (e, 0, 0)),
        ],
        out_specs=pl.BlockSpec((T, D), lambda e, f: (0, 0)),
        out_shape=jax.ShapeDtypeStruct((T, D), jnp.float32),
        scratch_shapes=[
            pltpu.VMEM((CP, D), jnp.float32),
            pltpu.VMEM((CP, D), jnp.float32),
        ],
    )(ft, fields, x, W1, b1.reshape(E, 1, F), W2, b2.reshape(E, 1, D))
    return y
